# baseline (speedup 1.0000x reference)
import jax
import jax.numpy as jnp
from jax import lax
from jax.experimental import pallas as pl
from jax.experimental.pallas import tpu as pltpu
from jax.experimental.pallas import tpu_sc as plsc

B = 16384
L = 200
NUM_W = 1000001
NUM_REL = 1000

NC, NS, LANES = 2, 16, 16
NW = NC * NS
ROWS_PER_W = B // NW
CHUNK = 64
NCHUNK = ROWS_PER_W // CHUNK
CW = CHUNK * L
FULL_VREGS = L // LANES
TAIL = L - FULL_VREGS * LANES


NGROUP = CHUNK // LANES
W_SLICE = 62504
NUM_W_PAD = W_SLICE * NS


N_STAGE = -(-W_SLICE // CW)
STAGE_TAIL = W_SLICE - (N_STAGE - 1) * CW
KH = 3


def _body(rules_hbm, rel_hbm, w_hbm, bias_hbm, out_hbm,
          rules_a, rules_b, vals_a, vals_b,
          bounce, bias_v, rel_v, out_acc,
          w_spmem, rsem_a, rsem_b, gsem_a, gsem_b, hsem, ssem):
  sid = lax.axis_index("s")
  wid = sid * NC + lax.axis_index("c")
  wbase = wid * ROWS_PER_W

  row_stride = lax.iota(jnp.int32, LANES) * L
  base_idx = [row_stride + g * (LANES * L) for g in range(NGROUP)]
  zero = jnp.zeros((LANES,), jnp.float32)

  rules_bufs = [rules_a, rules_b]
  vals_bufs = [vals_a, vals_b]
  rsem = [rsem_a, rsem_b]
  gsem = [gsem_a, gsem_b]
  stage_n = [CW] * (N_STAGE - 1) + [STAGE_TAIL]

  r_h, g_h = {}, {}
  stage_state = {}

  def issue_rules(c):
    p = c % 2
    r_h[c] = pltpu.async_copy(
        rules_hbm.at[pl.ds((wbase + c * CHUNK) * L, CW)], rules_bufs[p],
        rsem[p])

  def issue_gather(c):
    p = c % 2
    src = w_hbm if c < KH else w_spmem
    g_h[c] = pltpu.async_copy(src.at[rules_bufs[p]], vals_bufs[p], gsem[p])

  def issue_stage_read(k):
    stage_state["h"] = pltpu.async_copy(
        w_hbm.at[pl.ds(sid * W_SLICE + k * CW, stage_n[k])],
        bounce.at[pl.ds(0, stage_n[k])], hsem)

  def stage_step(k):
    stage_state["h"].wait()
    s = pltpu.async_copy(
        bounce.at[pl.ds(0, stage_n[k])],
        w_spmem.at[pl.ds(sid * W_SLICE + k * CW, stage_n[k])], ssem)
    s.wait()
    if k + 1 < N_STAGE:
      issue_stage_read(k + 1)

  issue_rules(0)
  issue_rules(1)
  issue_stage_read(0)
  pltpu.sync_copy(bias_hbm, bias_v)
  pltpu.sync_copy(rel_hbm.at[pl.ds(wbase, ROWS_PER_W)], rel_v)

  r_h[0].wait()
  issue_gather(0)
  stage_plan = {c: [] for c in range(NCHUNK)}
  for k in range(N_STAGE):
    stage_plan[min(k * KH // N_STAGE, KH - 1)].append(k)

  for c in range(NCHUNK):
    p = c % 2
    for k in stage_plan[c]:
      stage_step(k)
    if c + 1 < NCHUNK:
      r_h[c + 1].wait()
      if c + 1 == KH:
        plsc.subcore_barrier()
      issue_gather(c + 1)
    g_h[c].wait()
    if c + 2 < NCHUNK:
      issue_rules(c + 2)

    vals_ref = vals_bufs[p]

    def l_body(l, accs, vals_ref=vals_ref):
      return tuple(
          accs[g] + plsc.load_gather(vals_ref, [base_idx[g] + l])
          for g in range(NGROUP))

    accs = lax.fori_loop(0, L, l_body, (zero,) * NGROUP, unroll=8)
    for g in range(NGROUP):
      out_acc[pl.ds(c * CHUNK + g * LANES, LANES)] = accs[g]

  def bias_body(g, carry):
    idx = rel_v[pl.ds(g * LANES, LANES)]
    out_acc[pl.ds(g * LANES, LANES)] = (
        out_acc[pl.ds(g * LANES, LANES)] + plsc.load_gather(bias_v, [idx]))
    return carry

  lax.fori_loop(0, ROWS_PER_W // LANES, bias_body, 0)

  pltpu.sync_copy(out_acc, out_hbm.at[pl.ds(wbase, ROWS_PER_W)])


@jax.jit
def _run(rules_flat, relation, w_flat, bias_flat):
  mesh = plsc.VectorSubcoreMesh(
      core_axis_name="c", subcore_axis_name="s",
      num_cores=NC, num_subcores=NS)
  f = pl.kernel(
      _body,
      out_type=jax.ShapeDtypeStruct((B,), jnp.float32),
      mesh=mesh,
      compiler_params=pltpu.CompilerParams(needs_layout_passes=False),
      scratch_types=[
          pltpu.VMEM((CW,), jnp.int32),
          pltpu.VMEM((CW,), jnp.int32),
          pltpu.VMEM((CW,), jnp.float32),
          pltpu.VMEM((CW,), jnp.float32),
          pltpu.VMEM((CW,), jnp.float32),
          pltpu.VMEM((NUM_REL,), jnp.float32),
          pltpu.VMEM((ROWS_PER_W,), jnp.int32),
          pltpu.VMEM((ROWS_PER_W,), jnp.float32),
          pltpu.VMEM_SHARED((NUM_W_PAD,), jnp.float32),
          pltpu.SemaphoreType.DMA,
          pltpu.SemaphoreType.DMA,
          pltpu.SemaphoreType.DMA,
          pltpu.SemaphoreType.DMA,
          pltpu.SemaphoreType.DMA,
          pltpu.SemaphoreType.DMA,
      ],
  )
  return f(rules_flat, relation, w_flat, bias_flat)


def kernel(rules, relation, rules_weight, bias):
  rules_flat = rules.astype(jnp.int32).reshape(B * L)
  relation = relation.astype(jnp.int32)
  w_flat = jnp.concatenate([
      rules_weight.reshape(NUM_W),
      jnp.zeros((NUM_W_PAD - NUM_W,), jnp.float32)])
  bias_flat = bias.reshape(NUM_REL)
  out = _run(rules_flat, relation, w_flat, bias_flat)
  return out.reshape(B, 1)


# speedup vs baseline: 1.2215x; 1.2215x over previous
import jax
import jax.numpy as jnp
from jax import lax
from jax.experimental import pallas as pl
from jax.experimental.pallas import tpu as pltpu
from jax.experimental.pallas import tpu_sc as plsc

B = 16384
L = 200
NUM_W = 1000001
NUM_REL = 1000

NC, NS, LANES = 2, 16, 16
NW = NC * NS
ROWS_PER_W = B // NW
CHUNK = 64
NCHUNK = ROWS_PER_W // CHUNK
CW = CHUNK * L
FULL_VREGS = L // LANES
TAIL = L - FULL_VREGS * LANES


NGROUP = CHUNK // LANES
W_SLICE = 62504
NUM_W_PAD = W_SLICE * NS


SCW = 6400
N_STAGE = -(-W_SLICE // SCW)
STAGE_TAIL = W_SLICE - (N_STAGE - 1) * SCW


def _body(rules_hbm, rel_hbm, w_hbm, bias_hbm, out_hbm,
          rules_a, rules_b, vals_a, vals_b,
          bounce_a, bounce_b, bias_v, rel_v, out_acc,
          w_spmem, rsem_a, rsem_b, gsem_a, gsem_b, hsem, ssem):
  sid = lax.axis_index("s")
  wid = sid * NC + lax.axis_index("c")
  wbase = wid * ROWS_PER_W

  row_stride = lax.iota(jnp.int32, LANES) * L
  base_idx = [row_stride + g * (LANES * L) for g in range(NGROUP)]
  zero = jnp.zeros((LANES,), jnp.float32)

  rules_bufs = [rules_a, rules_b]
  vals_bufs = [vals_a, vals_b]
  rsem = [rsem_a, rsem_b]
  gsem = [gsem_a, gsem_b]
  bounce = [bounce_a, bounce_b]
  stage_n = [SCW] * (N_STAGE - 1) + [STAGE_TAIL]

  r_h, g_h, h_h = {}, {}, {}

  def issue_rules(c):
    p = c % 2
    r_h[c] = pltpu.async_copy(
        rules_hbm.at[pl.ds((wbase + c * CHUNK) * L, CW)], rules_bufs[p],
        rsem[p])

  def issue_gather(c):
    p = c % 2
    g_h[c] = pltpu.async_copy(w_spmem.at[rules_bufs[p]], vals_bufs[p],
                              gsem[p])

  def issue_stage_read(k):
    h_h[k] = pltpu.async_copy(
        w_hbm.at[pl.ds(sid * W_SLICE + k * SCW, stage_n[k])],
        bounce[k % 2].at[pl.ds(0, stage_n[k])], hsem)

  issue_rules(0)
  issue_rules(1)
  issue_stage_read(0)
  issue_stage_read(1)
  pltpu.sync_copy(bias_hbm, bias_v)
  pltpu.sync_copy(rel_hbm.at[pl.ds(wbase, ROWS_PER_W)], rel_v)

  for k in range(N_STAGE):
    h_h[k].wait()
    s = pltpu.async_copy(
        bounce[k % 2].at[pl.ds(0, stage_n[k])],
        w_spmem.at[pl.ds(sid * W_SLICE + k * SCW, stage_n[k])], ssem)
    s.wait()
    if k + 2 < N_STAGE:
      issue_stage_read(k + 2)

  plsc.subcore_barrier()

  r_h[0].wait()
  issue_gather(0)

  for c in range(NCHUNK):
    p = c % 2
    if c + 1 < NCHUNK:
      r_h[c + 1].wait()
      issue_gather(c + 1)
    g_h[c].wait()
    if c + 2 < NCHUNK:
      issue_rules(c + 2)

    vals_ref = vals_bufs[p]

    def l_body(l, accs, vals_ref=vals_ref):
      return tuple(
          accs[g] + plsc.load_gather(vals_ref, [base_idx[g] + l])
          for g in range(NGROUP))

    accs = lax.fori_loop(0, L, l_body, (zero,) * NGROUP, unroll=8)
    for g in range(NGROUP):
      out_acc[pl.ds(c * CHUNK + g * LANES, LANES)] = accs[g]

  def bias_body(g, carry):
    idx = rel_v[pl.ds(g * LANES, LANES)]
    out_acc[pl.ds(g * LANES, LANES)] = (
        out_acc[pl.ds(g * LANES, LANES)] + plsc.load_gather(bias_v, [idx]))
    return carry

  lax.fori_loop(0, ROWS_PER_W // LANES, bias_body, 0)

  pltpu.sync_copy(out_acc, out_hbm.at[pl.ds(wbase, ROWS_PER_W)])


@jax.jit
def _run(rules_flat, relation, w_flat, bias_flat):
  mesh = plsc.VectorSubcoreMesh(
      core_axis_name="c", subcore_axis_name="s",
      num_cores=NC, num_subcores=NS)
  f = pl.kernel(
      _body,
      out_type=jax.ShapeDtypeStruct((B,), jnp.float32),
      mesh=mesh,
      compiler_params=pltpu.CompilerParams(needs_layout_passes=False),
      scratch_types=[
          pltpu.VMEM((CW,), jnp.int32),
          pltpu.VMEM((CW,), jnp.int32),
          pltpu.VMEM((CW,), jnp.float32),
          pltpu.VMEM((CW,), jnp.float32),
          pltpu.VMEM((SCW,), jnp.float32),
          pltpu.VMEM((SCW,), jnp.float32),
          pltpu.VMEM((NUM_REL,), jnp.float32),
          pltpu.VMEM((ROWS_PER_W,), jnp.int32),
          pltpu.VMEM((ROWS_PER_W,), jnp.float32),
          pltpu.VMEM_SHARED((NUM_W_PAD,), jnp.float32),
          pltpu.SemaphoreType.DMA,
          pltpu.SemaphoreType.DMA,
          pltpu.SemaphoreType.DMA,
          pltpu.SemaphoreType.DMA,
          pltpu.SemaphoreType.DMA,
          pltpu.SemaphoreType.DMA,
      ],
  )
  return f(rules_flat, relation, w_flat, bias_flat)


def kernel(rules, relation, rules_weight, bias):
  rules_flat = rules.astype(jnp.int32).reshape(B * L)
  relation = relation.astype(jnp.int32)
  w_flat = jnp.concatenate([
      rules_weight.reshape(NUM_W),
      jnp.zeros((NUM_W_PAD - NUM_W,), jnp.float32)])
  bias_flat = bias.reshape(NUM_REL)
  out = _run(rules_flat, relation, w_flat, bias_flat)
  return out.reshape(B, 1)
